# Initial kernel scaffold; baseline (speedup 1.0000x reference)
import jax
import jax.numpy as jnp
from jax.experimental import pallas as pl


def kernel(action, holding, dominos):
    raise NotImplementedError("write your pallas kernel here")


# speedup vs baseline: 4.6291x; 4.6291x over previous
import functools

import jax
import jax.numpy as jnp
from jax import lax
from jax.experimental import pallas as pl
from jax.experimental.pallas import tpu as pltpu
from jax.experimental.pallas import tpu_sc as plsc

C = 4096
L = 16
NWORK = 32
STRIP = C // NWORK
CV = STRIP // L
R = 64
NCH = C // R
NBUF = 4
NG = NCH // NBUF


def _wm_body(dom, hexp, act, nh_out, ndom_out,
             buf0, buf1, buf2, buf3, hx, ast, nhb,
             in0, in1, in2, in3, out0, out1, out2, out3, hsem):
    bufs = (buf0, buf1, buf2, buf3)
    insems = (in0, in1, in2, in3)
    outsems = (out0, out1, out2, out3)
    wid = lax.axis_index("s") * 2 + lax.axis_index("c")
    j0 = wid * STRIP

    pltpu.async_copy(hexp, hx, hsem)
    pltpu.sync_copy(act.at[pl.ds(j0, STRIP)], ast)

    pltpu.async_copy(dom.at[pl.ds(0, R), pl.ds(j0, STRIP)], bufs[0], insems[0])
    pltpu.async_copy(dom.at[pl.ds(R, R), pl.ds(j0, STRIP)], bufs[1], insems[1])

    ap = [1.0 - ast[pl.ds(c * L, L)] for c in range(CV)]

    pltpu.make_async_copy(hexp, hx, hsem).wait()

    zeros = jnp.zeros((L,), jnp.float32)
    ms0 = tuple((zeros, zeros, zeros) for _ in range(CV))

    def make_row_body(buf, kbase):
        def row_body(r, ms):
            hv = hx[kbase + r, :]
            hp = 1.0 - hv
            new = []
            for c in range(CV):
                sl = pl.ds(c * L, L)
                d = buf[r, sl]
                p1 = d * ap[c]
                p2 = d * hp
                buf[r, sl] = p1 + p2 - p1 * p2
                v = d - p2
                m1, m2, m3 = ms[c]
                n1 = jnp.maximum(m1, v)
                x = jnp.minimum(m1, v)
                n2 = jnp.maximum(m2, x)
                x = jnp.minimum(m2, x)
                n3 = jnp.maximum(m3, x)
                new.append((n1, n2, n3))
            return tuple(new)
        return row_body

    def group(g, ms):
        for b in range(NBUF):
            k = NBUF * g + b
            pltpu.make_async_copy(
                dom.at[pl.ds(k * R, R), pl.ds(j0, STRIP)],
                bufs[b], insems[b]).wait()
            b2 = (b + 2) % NBUF

            def drain(b2=b2):
                pltpu.make_async_copy(
                    bufs[b2],
                    ndom_out.at[pl.ds(0, R), pl.ds(j0, STRIP)],
                    outsems[b2]).wait()

            def prefetch(k=k, b2=b2):
                pltpu.async_copy(
                    dom.at[pl.ds((k + 2) * R, R), pl.ds(j0, STRIP)],
                    bufs[b2], insems[b2])

            if b < 2:
                pl.when(g >= 1)(drain)
                prefetch()
            else:
                drain()
                pl.when(g < NG - 1)(prefetch)

            ms = lax.fori_loop(0, R, make_row_body(bufs[b], k * R), ms)
            pltpu.async_copy(
                bufs[b],
                ndom_out.at[pl.ds(k * R, R), pl.ds(j0, STRIP)],
                outsems[b])
        return ms

    ms = lax.fori_loop(0, NG, group, ms0)

    for b in (2, 3):
        pltpu.make_async_copy(
            bufs[b], ndom_out.at[pl.ds(0, R), pl.ds(j0, STRIP)],
            outsems[b]).wait()

    for c in range(CV):
        m1, m2, m3 = ms[c]
        av = ast[pl.ds(c * L, L)]
        t = (1.0 - m1 * av) * ((1.0 - m2 * av) * (1.0 - m3 * av))
        nhb[pl.ds(c * L, L)] = 1.0 - t
    pltpu.sync_copy(nhb, nh_out.at[pl.ds(j0, STRIP)])


_wm_call = functools.partial(
    pl.kernel,
    mesh=plsc.VectorSubcoreMesh(core_axis_name="c", subcore_axis_name="s"),
    compiler_params=pltpu.CompilerParams(use_tc_tiling_on_sc=False),
    out_type=[
        jax.ShapeDtypeStruct((C,), jnp.float32),
        jax.ShapeDtypeStruct((C, C), jnp.float32),
    ],
    scratch_types=[
        pltpu.VMEM((R, STRIP), jnp.float32),
        pltpu.VMEM((R, STRIP), jnp.float32),
        pltpu.VMEM((R, STRIP), jnp.float32),
        pltpu.VMEM((R, STRIP), jnp.float32),
        pltpu.VMEM((C, L), jnp.float32),
        pltpu.VMEM((STRIP,), jnp.float32),
        pltpu.VMEM((STRIP,), jnp.float32),
        pltpu.SemaphoreType.DMA,
        pltpu.SemaphoreType.DMA,
        pltpu.SemaphoreType.DMA,
        pltpu.SemaphoreType.DMA,
        pltpu.SemaphoreType.DMA,
        pltpu.SemaphoreType.DMA,
        pltpu.SemaphoreType.DMA,
        pltpu.SemaphoreType.DMA,
        pltpu.SemaphoreType.DMA,
    ],
)(_wm_body)


def kernel(action, holding, dominos):
    dom = dominos.reshape(C, C)
    hexp = jnp.broadcast_to(holding[:, None], (C, L))
    nh, ndom = _wm_call(dom, hexp, action)
    return nh, ndom.reshape(-1)
